# baseline (speedup 1.0000x reference)
import functools

import jax
import jax.numpy as jnp
from jax import lax
from jax.experimental import pallas as pl
from jax.experimental.pallas import tpu as pltpu
from jax.experimental.pallas import tpu_sc as plsc

EMBED_DIM = 64
SEQ = 200
BATCH = 4096

_info = plsc.get_sparse_core_info()
_NC = _info.num_cores
_NS = _info.num_subcores
_NW = _NC * _NS

_BB = BATCH // _NW
_ET = EMBED_DIM // 8


def _make_embed():
    mesh = plsc.VectorSubcoreMesh(core_axis_name="c", subcore_axis_name="s")

    @functools.partial(
        pl.kernel,
        mesh=mesh,
        out_type=jax.ShapeDtypeStruct((SEQ, _ET, _NW, 8, _BB), jnp.float32),
        scratch_types=[
            pltpu.VMEM((2, _BB), jnp.int32),
            pltpu.VMEM((2, _BB, EMBED_DIM), jnp.float32),
            pltpu.VMEM((2, _ET, 8, _BB), jnp.float32),
            pltpu.SemaphoreType.DMA,
            pltpu.SemaphoreType.DMA,
            pltpu.SemaphoreType.DMA,
            pltpu.SemaphoreType.DMA,
            pltpu.SemaphoreType.DMA,
            pltpu.SemaphoreType.DMA,
        ],
        compiler_params=pltpu.CompilerParams(
            use_tc_tiling_on_sc=False, needs_layout_passes=False
        ),
    )
    def embed(table_hbm, idx_hbm, out_hbm, idx_v, rows_v, tr_v,
              sem_i0, sem_i1, sem_g0, sem_g1, sem_o0, sem_o1):
        sem_i = [sem_i0, sem_i1]
        sem_g = [sem_g0, sem_g1]
        sem_o = [sem_o0, sem_o1]
        wid = lax.axis_index("s") * _NC + lax.axis_index("c")
        b0 = wid * _BB

        def start_idx(t, b):
            pltpu.async_copy(
                idx_hbm.at[t].at[pl.ds(b0, _BB)], idx_v.at[b], sem_i[b]
            )

        def wait_idx(b):
            pltpu.make_async_copy(
                idx_hbm.at[0].at[pl.ds(b0, _BB)], idx_v.at[b], sem_i[b]
            ).wait()

        def start_gather(b):
            pltpu.async_copy(
                table_hbm.at[idx_v.at[b]], rows_v.at[b], sem_g[b]
            )

        def wait_gather(b):
            pltpu.make_async_copy(
                table_hbm.at[idx_v.at[b]], rows_v.at[b], sem_g[b]
            ).wait()

        def start_out(t, b):
            for e_t in range(_ET):
                pltpu.async_copy(
                    tr_v.at[b].at[e_t],
                    out_hbm.at[t].at[e_t].at[wid],
                    sem_o[b],
                )

        def wait_out(b):
            for e_t in range(_ET):
                pltpu.make_async_copy(
                    tr_v.at[b].at[e_t],
                    out_hbm.at[0].at[e_t].at[wid],
                    sem_o[b],
                ).wait()

        def transpose(b):
            lane = lax.iota(jnp.int32, 16)

            def body(e, carry):
                e_t = e // 8
                e_s = e % 8
                col = jnp.full((16,), e, dtype=jnp.int32)
                for k in range(_BB // 16):
                    vals = plsc.load_gather(
                        rows_v.at[b], [lane + (16 * k), col]
                    )
                    tr_v[b, e_t, e_s, pl.ds(16 * k, 16)] = vals
                return carry

            lax.fori_loop(0, EMBED_DIM, body, 0)

        def process(t, b, first, prefetch, fire_next):
            wait_gather(b)
            if prefetch:
                start_idx(t + 2, b)
            if not first:
                wait_out(b)
            if fire_next:
                wait_idx(1 - b)
                start_gather(1 - b)
            transpose(b)
            start_out(t, b)

        start_idx(0, 0)
        start_idx(1, 1)
        wait_idx(0)
        start_gather(0)

        process(0, 0, first=True, prefetch=True, fire_next=True)
        process(1, 1, first=True, prefetch=True, fire_next=True)

        def body(g, carry):
            t = 2 * g
            process(t, 0, first=False, prefetch=True, fire_next=True)
            process(t + 1, 1, first=False, prefetch=True, fire_next=True)
            return carry

        lax.fori_loop(1, SEQ // 2 - 1, body, 0)

        t = SEQ - 2
        process(t, 0, first=False, prefetch=False, fire_next=True)
        process(t + 1, 1, first=False, prefetch=False, fire_next=False)

        wait_out(0)
        wait_out(1)

    return embed


def kernel(tokens, table):
    tokens_t = tokens.T
    out5 = _make_embed()(table, tokens_t)
    out = out5.transpose((2, 4, 0, 1, 3)).reshape(BATCH, SEQ, EMBED_DIM)
    return out


# speedup vs baseline: 1.0017x; 1.0017x over previous
import functools

import jax
import jax.numpy as jnp
from jax import lax
from jax.experimental import pallas as pl
from jax.experimental.pallas import tpu as pltpu
from jax.experimental.pallas import tpu_sc as plsc

EMBED_DIM = 64
SEQ = 200
BATCH = 4096

_info = plsc.get_sparse_core_info()
_NC = _info.num_cores
_NS = _info.num_subcores
_NW = _NC * _NS

_BB = BATCH // _NW
_ET = EMBED_DIM // 8
_NBUF = 4
_J = 2
_JR = _BB // _J


def _make_embed():
    mesh = plsc.VectorSubcoreMesh(core_axis_name="c", subcore_axis_name="s")

    @functools.partial(
        pl.kernel,
        mesh=mesh,
        out_type=jax.ShapeDtypeStruct((SEQ, _ET, _NW, 8, _BB), jnp.float32),
        scratch_types=[
            pltpu.VMEM((_NBUF, _BB), jnp.int32),
            pltpu.VMEM((_NBUF, _BB, EMBED_DIM), jnp.float32),
            pltpu.VMEM((_NBUF, _ET, 8, _BB), jnp.float32),
        ]
        + [pltpu.SemaphoreType.DMA] * (3 * _NBUF),
        compiler_params=pltpu.CompilerParams(
            use_tc_tiling_on_sc=False, needs_layout_passes=False
        ),
    )
    def embed(table_hbm, idx_hbm, out_hbm, idx_v, rows_v, tr_v, *sems):
        sem_i = sems[0:_NBUF]
        sem_g = sems[_NBUF:2 * _NBUF]
        sem_o = sems[2 * _NBUF:3 * _NBUF]
        wid = lax.axis_index("s") * _NC + lax.axis_index("c")
        b0 = wid * _BB

        lanes = [lax.iota(jnp.int32, 16) + (16 * k) for k in range(_BB // 16)]

        def start_idx(t, s):
            pltpu.async_copy(
                idx_hbm.at[t].at[pl.ds(b0, _BB)], idx_v.at[s], sem_i[s]
            )

        def wait_idx(s):
            pltpu.make_async_copy(
                idx_hbm.at[0].at[pl.ds(b0, _BB)], idx_v.at[s], sem_i[s]
            ).wait()

        def start_gather(s):
            for j in range(_J):
                pltpu.async_copy(
                    table_hbm.at[idx_v.at[s].at[pl.ds(j * _JR, _JR)]],
                    rows_v.at[s].at[pl.ds(j * _JR, _JR)],
                    sem_g[s],
                )

        def wait_gather(s):
            for j in range(_J):
                pltpu.make_async_copy(
                    table_hbm.at[idx_v.at[s].at[pl.ds(j * _JR, _JR)]],
                    rows_v.at[s].at[pl.ds(j * _JR, _JR)],
                    sem_g[s],
                ).wait()

        def start_out(t, s):
            for e_t in range(_ET):
                pltpu.async_copy(
                    tr_v.at[s].at[e_t],
                    out_hbm.at[t].at[e_t].at[wid],
                    sem_o[s],
                )

        def wait_out(s):
            for e_t in range(_ET):
                pltpu.make_async_copy(
                    tr_v.at[s].at[e_t],
                    out_hbm.at[0].at[e_t].at[wid],
                    sem_o[s],
                ).wait()

        def transpose(s):
            def body(e, carry):
                e_t = e // 8
                e_s = e % 8
                col = jnp.full((16,), e, dtype=jnp.int32)
                for k in range(_BB // 16):
                    vals = plsc.load_gather(rows_v.at[s], [lanes[k], col])
                    tr_v[s, e_t, e_s, pl.ds(16 * k, 16)] = vals
                return carry

            lax.fori_loop(0, EMBED_DIM, body, 0)

        def process(t, s, first, prefetch, fire_next):
            wait_gather(s)
            if prefetch:
                start_idx(t + _NBUF, s)
            if fire_next:
                wait_idx((s + _NBUF - 1) % _NBUF)
                start_gather((s + _NBUF - 1) % _NBUF)
            if not first:
                wait_out(s)
            transpose(s)
            start_out(t, s)

        for s in range(_NBUF):
            start_idx(s, s)
        for s in range(_NBUF - 1):
            wait_idx(s)
            start_gather(s)

        for t in range(_NBUF):
            process(t, t, first=True, prefetch=True, fire_next=True)

        def body(g, carry):
            t = _NBUF * g
            for s in range(_NBUF):
                process(t + s, s, first=False, prefetch=True, fire_next=True)
            return carry

        lax.fori_loop(1, SEQ // _NBUF - 1, body, 0)

        for s in range(_NBUF):
            t = SEQ - _NBUF + s
            process(t, s, first=False, prefetch=False,
                    fire_next=(s == 0))

        for s in range(_NBUF):
            wait_out(s)

    return embed


def kernel(tokens, table):
    tokens_t = tokens.T
    out5 = _make_embed()(table, tokens_t)
    out = out5.transpose((2, 4, 0, 1, 3)).reshape(BATCH, SEQ, EMBED_DIM)
    return out
